# Initial kernel scaffold; baseline (speedup 1.0000x reference)
import jax
import jax.numpy as jnp
from jax import lax
from jax.experimental import pallas as pl
from jax.experimental.pallas import tpu as pltpu
from jax.experimental.pallas import tpu_sc as plsc

B = 32
C = 96
H = 96
W = 96
HP = H + 2
WP = W + 2
IN_IMG = H * W
OUT_IMG = HP * WP
CH_PER = 2
NSTEP = C // CH_PER
IN_CHUNK = CH_PER * IN_IMG
OUT_CHUNK = CH_PER * OUT_IMG

NC = 2
NS = 16


def _body(x_hbm, out_hbm, in_v, out_v, pad_v):
    b = lax.axis_index("s") * NC + lax.axis_index("c")

    iota = lax.iota(jnp.int32, 16)
    iota96 = iota * W
    iota98 = iota * WP
    third = jnp.float32(1.0 / 3.0)

    one = jnp.float32(1.0)
    zero = jnp.float32(0.0)
    pb = b % 16
    tz = jnp.where(pb < 4, zero, one)
    bz = jnp.where(pb >= 12, zero, one)
    lz = jnp.where(b % 4 == 0, zero, one)
    rz = jnp.where(b % 4 == 3, zero, one)

    pad_v[pl.ds(W, 16)] = jnp.zeros((16,), jnp.float32)

    def step(t, carry):
        in_off = b * (C * IN_IMG) + t * IN_CHUNK
        out_off = b * (C * OUT_IMG) + t * OUT_CHUNK
        pltpu.sync_copy(x_hbm.at[pl.ds(in_off, IN_CHUNK)], in_v)

        for img in range(CH_PER):
            ib = img * IN_IMG
            ob = img * OUT_IMG

            def row(h, c2):
                src = ib + h * W
                dst = ob + (WP + 1) + h * WP
                for k in range(W // 16):
                    v = in_v[pl.ds(src + k * 16, 16)]
                    plsc.store_scatter(out_v, [dst + k * 16 + iota], v)
                return c2

            lax.fori_loop(0, H, row, 0, unroll=2)

            for r0 in range(0, H, 16):
                idx = (ib + r0 * W) + iota96
                la = plsc.load_gather(in_v, [idx])
                lb = plsc.load_gather(in_v, [idx + 1])
                lc = plsc.load_gather(in_v, [idx + 2])
                lv = (la + lb + lc) * third * lz
                oidx = (ob + (r0 + 1) * WP) + iota98
                plsc.store_scatter(out_v, [oidx], lv)
                ra = plsc.load_gather(in_v, [idx + (W - 3)])
                rb = plsc.load_gather(in_v, [idx + (W - 2)])
                rc = plsc.load_gather(in_v, [idx + (W - 1)])
                rv = (ra + rb + rc) * third * rz
                plsc.store_scatter(out_v, [oidx + (WP - 1)], rv)

            for src_row, obase, mz in (
                (0, ob + 1, tz),
                ((H - 1) * W, ob + (HP - 1) * WP + 1, bz),
            ):
                for k in range(W // 16):
                    pad_v[pl.ds(k * 16, 16)] = in_v[pl.ds(ib + src_row + k * 16, 16)]
                for k in range(W // 16):
                    j = k * 16
                    ta = pad_v[pl.ds(j, 16)]
                    tb = plsc.load_gather(pad_v, [j + 1 + iota])
                    tc = plsc.load_gather(pad_v, [j + 2 + iota])
                    tv = (ta + tb + tc) * third * mz
                    plsc.store_scatter(out_v, [obase + j + iota], tv)

            csrc = ib + jnp.where(iota < 2, 0, (H - 1) * W) + \
                jnp.where(iota % 2 == 1, W - 1, 0)
            cv = plsc.load_gather(in_v, [csrc])
            cm = jnp.where(iota < 2, tz, bz) * jnp.where(iota % 2 == 0, lz, rz)
            cdst = ob + jnp.where(iota < 2, 0, (HP - 1) * WP) + \
                jnp.where(iota % 2 == 1, WP - 1, 0)
            plsc.store_scatter(out_v, [cdst], cv * cm, mask=iota < 4)

        pltpu.sync_copy(out_v, out_hbm.at[pl.ds(out_off, OUT_CHUNK)])
        return carry

    lax.fori_loop(0, NSTEP, step, 0)


@jax.jit
def kernel(x):
    mesh = plsc.VectorSubcoreMesh(
        core_axis_name="c", subcore_axis_name="s",
        num_cores=NC, num_subcores=NS,
    )
    run = pl.kernel(
        _body,
        out_type=jax.ShapeDtypeStruct((B * C * OUT_IMG,), jnp.float32),
        mesh=mesh,
        scratch_types=[
            pltpu.VMEM((IN_CHUNK,), jnp.float32),
            pltpu.VMEM((OUT_CHUNK,), jnp.float32),
            pltpu.VMEM((112,), jnp.float32),
        ],
    )
    y = run(x.reshape(-1))
    return y.reshape(B, C, HP, WP)


# speedup vs baseline: 5.6989x; 5.6989x over previous
import jax
import jax.numpy as jnp
from jax import lax
from jax.experimental import pallas as pl
from jax.experimental.pallas import tpu as pltpu
from jax.experimental.pallas import tpu_sc as plsc

B = 32
C = 96
H = 96
W = 96
HP = H + 2
WP = W + 2
IN_IMG = H * W
OUT_IMG = HP * WP
CH_PER = 2
NSTEP = C // CH_PER
IN_CHUNK = CH_PER * IN_IMG
OUT_CHUNK = CH_PER * OUT_IMG

NC = 2
NS = 16


def _body(x_hbm, out_hbm, in_v, out_v, pad_v):
    b = lax.axis_index("s") * NC + lax.axis_index("c")

    iota = lax.iota(jnp.int32, 16)
    iota96 = iota * W
    iota98 = iota * WP
    third = jnp.float32(1.0 / 3.0)

    one = jnp.float32(1.0)
    zero = jnp.float32(0.0)
    pb = b % 16
    tz = jnp.where(pb < 4, zero, one)
    bz = jnp.where(pb >= 12, zero, one)
    lz = jnp.where(b % 4 == 0, zero, one)
    rz = jnp.where(b % 4 == 3, zero, one)

    pad_v[pl.ds(W, 16)] = jnp.zeros((16,), jnp.float32)

    def step(t, carry):
        in_off = b * (C * IN_IMG) + t * IN_CHUNK
        out_off = b * (C * OUT_IMG) + t * OUT_CHUNK
        pltpu.sync_copy(x_hbm.at[pl.ds(in_off, IN_CHUNK)], in_v)

        for img in range(CH_PER):
            ib = img * IN_IMG
            ob = img * OUT_IMG

            def row(h, c2):
                src = ib + h * W
                dst = ob + (WP + 1) + h * WP
                for k in range(W // 16):
                    v = in_v[pl.ds(src + k * 16, 16)]
                    plsc.store_scatter(out_v, [dst + k * 16 + iota], v)
                return c2

            lax.fori_loop(0, H, row, 0, unroll=2)

            for r0 in range(0, H, 16):
                idx = (ib + r0 * W) + iota96
                la = plsc.load_gather(in_v, [idx])
                lb = plsc.load_gather(in_v, [idx + 1])
                lc = plsc.load_gather(in_v, [idx + 2])
                lv = (la + lb + lc) * third * lz
                oidx = (ob + (r0 + 1) * WP) + iota98
                plsc.store_scatter(out_v, [oidx], lv)
                ra = plsc.load_gather(in_v, [idx + (W - 3)])
                rb = plsc.load_gather(in_v, [idx + (W - 2)])
                rc = plsc.load_gather(in_v, [idx + (W - 1)])
                rv = (ra + rb + rc) * third * rz
                plsc.store_scatter(out_v, [oidx + (WP - 1)], rv)

            for src_row, obase, mz in (
                (0, ob + 1, tz),
                ((H - 1) * W, ob + (HP - 1) * WP + 1, bz),
            ):
                for k in range(W // 16):
                    pad_v[pl.ds(k * 16, 16)] = in_v[pl.ds(ib + src_row + k * 16, 16)]
                for k in range(W // 16):
                    j = k * 16
                    ta = pad_v[pl.ds(j, 16)]
                    tb = plsc.load_gather(pad_v, [j + 1 + iota])
                    tc = plsc.load_gather(pad_v, [j + 2 + iota])
                    tv = (ta + tb + tc) * third * mz
                    plsc.store_scatter(out_v, [obase + j + iota], tv)

            csrc = ib + jnp.where(iota < 2, 0, (H - 1) * W) + \
                jnp.where(iota % 2 == 1, W - 1, 0)
            cv = plsc.load_gather(in_v, [csrc])
            cm = jnp.where(iota < 2, tz, bz) * jnp.where(iota % 2 == 0, lz, rz)
            cdst = ob + jnp.where(iota < 2, 0, (HP - 1) * WP) + \
                jnp.where(iota % 2 == 1, WP - 1, 0)
            plsc.store_scatter(out_v, [cdst], cv * cm, mask=iota < 4)

        pltpu.sync_copy(out_v, out_hbm.at[pl.ds(out_off, OUT_CHUNK)])
        return carry

    lax.fori_loop(0, NSTEP, step, 0)


@jax.jit
def kernel(x):
    mesh = plsc.VectorSubcoreMesh(
        core_axis_name="c", subcore_axis_name="s",
        num_cores=NC, num_subcores=NS,
    )
    run = pl.kernel(
        _body,
        out_type=jax.ShapeDtypeStruct((B * C * OUT_IMG,), jnp.float32),
        mesh=mesh,
        compiler_params=pltpu.CompilerParams(needs_layout_passes=False),
        scratch_types=[
            pltpu.VMEM((IN_CHUNK,), jnp.float32),
            pltpu.VMEM((OUT_CHUNK,), jnp.float32),
            pltpu.VMEM((112,), jnp.float32),
        ],
    )
    y = run(x.reshape(-1))
    return y.reshape(B, C, HP, WP)


# speedup vs baseline: 7.1481x; 1.2543x over previous
import jax
import jax.numpy as jnp
from jax import lax
from jax.experimental import pallas as pl
from jax.experimental.pallas import tpu as pltpu
from jax.experimental.pallas import tpu_sc as plsc

B = 32
C = 96
H = 96
W = 96
HP = H + 2
WP = W + 2
IN_IMG = H * W
OUT_IMG = HP * WP
CH_PER = 4
NSTEP = C // CH_PER
IN_CHUNK = CH_PER * IN_IMG
OUT_CHUNK = CH_PER * OUT_IMG

NC = 2
NS = 16


def _body(x_hbm, out_hbm, in_v, out_v, pad_v):
    b = lax.axis_index("s") * NC + lax.axis_index("c")

    iota = lax.iota(jnp.int32, 16)
    iota96 = iota * W
    iota98 = iota * WP
    third = jnp.float32(1.0 / 3.0)

    one = jnp.float32(1.0)
    zero = jnp.float32(0.0)
    pb = b % 16
    tz = jnp.where(pb < 4, zero, one)
    bz = jnp.where(pb >= 12, zero, one)
    lz = jnp.where(b % 4 == 0, zero, one)
    rz = jnp.where(b % 4 == 3, zero, one)

    pad_v[pl.ds(W, 16)] = jnp.zeros((16,), jnp.float32)

    def step(t, carry):
        in_off = b * (C * IN_IMG) + t * IN_CHUNK
        out_off = b * (C * OUT_IMG) + t * OUT_CHUNK
        pltpu.sync_copy(x_hbm.at[pl.ds(in_off, IN_CHUNK)], in_v)

        @plsc.parallel_loop(0, CH_PER * H, 1, unroll=4)
        def row(r):
            img = r // H
            h = r % H
            src = img * IN_IMG + h * W
            dst = img * OUT_IMG + (WP + 1) + h * WP
            for k in range(W // 16):
                v = in_v[pl.ds(src + k * 16, 16)]
                out_v[pl.ds(dst + k * 16, 16)] = v

        for img in range(CH_PER):
            ib = img * IN_IMG
            ob = img * OUT_IMG

            for r0 in range(0, H, 16):
                idx = (ib + r0 * W) + iota96
                la = plsc.load_gather(in_v, [idx])
                lb = plsc.load_gather(in_v, [idx + 1])
                lc = plsc.load_gather(in_v, [idx + 2])
                lv = (la + lb + lc) * third * lz
                oidx = (ob + (r0 + 1) * WP) + iota98
                plsc.store_scatter(out_v, [oidx], lv)
                ra = plsc.load_gather(in_v, [idx + (W - 3)])
                rb = plsc.load_gather(in_v, [idx + (W - 2)])
                rc = plsc.load_gather(in_v, [idx + (W - 1)])
                rv = (ra + rb + rc) * third * rz
                plsc.store_scatter(out_v, [oidx + (WP - 1)], rv)

            for src_row, obase, mz in (
                (0, ob + 1, tz),
                ((H - 1) * W, ob + (HP - 1) * WP + 1, bz),
            ):
                for k in range(W // 16):
                    pad_v[pl.ds(k * 16, 16)] = in_v[pl.ds(ib + src_row + k * 16, 16)]
                for k in range(W // 16):
                    j = k * 16
                    ta = pad_v[pl.ds(j, 16)]
                    tb = plsc.load_gather(pad_v, [j + 1 + iota])
                    tc = plsc.load_gather(pad_v, [j + 2 + iota])
                    tv = (ta + tb + tc) * third * mz
                    plsc.store_scatter(out_v, [obase + j + iota], tv)

            csrc = ib + jnp.where(iota < 2, 0, (H - 1) * W) + \
                jnp.where(iota % 2 == 1, W - 1, 0)
            cv = plsc.load_gather(in_v, [csrc])
            cm = jnp.where(iota < 2, tz, bz) * jnp.where(iota % 2 == 0, lz, rz)
            cdst = ob + jnp.where(iota < 2, 0, (HP - 1) * WP) + \
                jnp.where(iota % 2 == 1, WP - 1, 0)
            plsc.store_scatter(out_v, [cdst], cv * cm, mask=iota < 4)

        pltpu.sync_copy(out_v, out_hbm.at[pl.ds(out_off, OUT_CHUNK)])
        return carry

    lax.fori_loop(0, NSTEP, step, 0)


@jax.jit
def kernel(x):
    mesh = plsc.VectorSubcoreMesh(
        core_axis_name="c", subcore_axis_name="s",
        num_cores=NC, num_subcores=NS,
    )
    run = pl.kernel(
        _body,
        out_type=jax.ShapeDtypeStruct((B * C * OUT_IMG,), jnp.float32),
        mesh=mesh,
        compiler_params=pltpu.CompilerParams(needs_layout_passes=False),
        scratch_types=[
            pltpu.VMEM((IN_CHUNK,), jnp.float32),
            pltpu.VMEM((OUT_CHUNK,), jnp.float32),
            pltpu.VMEM((112,), jnp.float32),
        ],
    )
    y = run(x.reshape(-1))
    return y.reshape(B, C, HP, WP)
